# Initial kernel scaffold; baseline (speedup 1.0000x reference)
import jax
import jax.numpy as jnp
from jax.experimental import pallas as pl


def kernel(z, emb_weight):
    raise NotImplementedError("write your pallas kernel here")


# speedup vs baseline: 1.0247x; 1.0247x over previous
import functools

import jax
import jax.numpy as jnp
from jax import lax
from jax.experimental import pallas as pl
from jax.experimental.pallas import tpu as pltpu
from jax.experimental.pallas import tpu_sc as plsc

_K = 8192
_D = 32
_CF = 0.1
_TOK = 4096
_TOK_BLK = 1024
_K_BLK = 1024
_NC = 2
_NS = 16
_NW = _NC * _NS
_B_W = _TOK // _NW
_BIG = 2**30


def _tc_body(zn_ref, z_ref, emb_ref, idx_ref, loss_ref):
    t = pl.program_id(0)
    flat = z_ref[0]
    zn = zn_ref[0]

    def chunk_step(i, carry):
        best_d, best_i = carry
        chunk = emb_ref[pl.ds(i * _K_BLK, _K_BLK), :]
        en = jnp.sum(chunk * chunk, axis=1, keepdims=True)
        m = lax.dot_general(chunk, flat, (((1,), (1,)), ((), ())))
        d = (zn + en) - 2.0 * m
        cmin = jnp.min(d, axis=0, keepdims=True)
        iota = lax.broadcasted_iota(jnp.int32, d.shape, 0) + i * _K_BLK
        cidx = jnp.min(jnp.where(d == cmin, iota, _BIG), axis=0, keepdims=True)
        upd = cmin < best_d
        return jnp.where(upd, cmin, best_d), jnp.where(upd, cidx, best_i)

    init = (jnp.full((1, _TOK_BLK), jnp.inf, jnp.float32),
            jnp.full((1, _TOK_BLK), _BIG, jnp.int32))
    best_d, best_i = lax.fori_loop(0, _K // _K_BLK, chunk_step, init)
    idx_ref[...] = best_i[None]
    partial = jnp.sum(best_d) * (_CF / (_TOK * _D))

    @pl.when(t == 0)
    def _():
        loss_ref[0, 0] = partial

    @pl.when(t != 0)
    def _():
        loss_ref[0, 0] = loss_ref[0, 0] + partial


def _tc_argmin(zn, z, emb):
    grid = _TOK // _TOK_BLK
    return pl.pallas_call(
        _tc_body,
        grid=(grid,),
        in_specs=[
            pl.BlockSpec((1, 1, _TOK_BLK), lambda t: (t, 0, 0)),
            pl.BlockSpec((1, _TOK_BLK, _D), lambda t: (t, 0, 0)),
            pl.BlockSpec((_K, _D), lambda t: (0, 0)),
        ],
        out_specs=[
            pl.BlockSpec((1, 1, _TOK_BLK), lambda t: (t, 0, 0)),
            pl.BlockSpec(memory_space=pltpu.SMEM, block_shape=(1, 1),
                         index_map=lambda t: (0, 0)),
        ],
        out_shape=[
            jax.ShapeDtypeStruct((grid, 1, _TOK_BLK), jnp.int32),
            jax.ShapeDtypeStruct((1, 1), jnp.float32),
        ],
    )(zn, z, emb)


def _sc_gather_body(idx_hbm, z_hbm, emb_hbm, out_hbm, idx_v, z_v, rows_v,
                    out_v, sem):
    wid = lax.axis_index("s") * _NC + lax.axis_index("c")
    base = wid * _B_W
    pltpu.sync_copy(idx_hbm.at[pl.ds(base, _B_W)], idx_v)
    pltpu.sync_copy(z_hbm.at[pl.ds(base, _B_W)], z_v)
    pltpu.async_copy(emb_hbm.at[idx_v], rows_v, sem).wait()

    def row(r, carry):
        for c in (0, 16):
            q = rows_v[r, pl.ds(c, 16)]
            zz = z_v[r, pl.ds(c, 16)]
            out_v[r, pl.ds(c, 16)] = zz + (q - zz)
        return carry

    lax.fori_loop(0, _B_W, row, 0)
    pltpu.sync_copy(out_v, out_hbm.at[pl.ds(base, _B_W)])


@functools.lru_cache(maxsize=None)
def _make_sc_gather():
    return pl.kernel(
        _sc_gather_body,
        mesh=plsc.VectorSubcoreMesh(core_axis_name="c", subcore_axis_name="s",
                                    num_cores=_NC, num_subcores=_NS),
        out_type=jax.ShapeDtypeStruct((_TOK, _D), jnp.float32),
        scratch_types=[
            pltpu.VMEM((_B_W,), jnp.int32),
            pltpu.VMEM((_B_W, _D), jnp.float32),
            pltpu.VMEM((_B_W, _D), jnp.float32),
            pltpu.VMEM((_B_W, _D), jnp.float32),
            pltpu.SemaphoreType.DMA,
        ],
        compiler_params=pltpu.CompilerParams(use_tc_tiling_on_sc=False),
    )


def kernel(z, emb_weight):
    flat = z.reshape(_TOK, _D)
    zn = (flat ** 2).sum(axis=-1).reshape(_TOK // _TOK_BLK, 1, _TOK_BLK)
    idx2d, loss = _tc_argmin(zn, z, emb_weight)
    idx = idx2d.reshape(_TOK)
    qst = _make_sc_gather()(idx, flat, emb_weight)
    return (qst.reshape(z.shape), idx.reshape(_TOK, 1), loss[0, 0])
